# baseline (speedup 1.0000x reference)
import functools

import jax
import jax.numpy as jnp
from jax import lax
from jax.experimental import pallas as pl
from jax.experimental.pallas import tpu as pltpu
from jax.experimental.pallas import tpu_sc as plsc

B = 16384
DIM = 128
GAMMA = 12.0

NC = 2
NS = 16
L = 16
NW = NC * NS
BPW = B // NW
CH = 128
NCH = BPW // CH

_mesh = plsc.VectorSubcoreMesh(core_axis_name="c", subcore_axis_name="s")


@functools.partial(
    pl.kernel,
    mesh=_mesh,
    out_type=jax.ShapeDtypeStruct((B,), jnp.float32),
    scratch_types=[
        pltpu.VMEM((NCH, CH), jnp.int32),
        pltpu.VMEM((NCH, CH), jnp.int32),
        pltpu.VMEM((NCH, CH), jnp.int32),
        pltpu.VMEM((CH, DIM), jnp.float32),
        pltpu.VMEM((CH, DIM), jnp.float32),
        pltpu.VMEM((CH, DIM), jnp.float32),
        pltpu.VMEM((CH, DIM), jnp.float32),
        pltpu.VMEM((CH, DIM), jnp.float32),
        pltpu.VMEM((CH, DIM), jnp.float32),
        pltpu.VMEM((BPW,), jnp.float32),
        pltpu.SemaphoreType.DMA,
        pltpu.SemaphoreType.DMA,
    ],
    compiler_params=pltpu.CompilerParams(needs_layout_passes=False),
)
def _sc_score(hi_hbm, ri_hbm, ti_hbm, ent_hbm, rel_hbm, out_hbm,
              hi_v, ri_v, ti_v, hb0, hb1, rb0, rb1, tb0, tb1, ob,
              sem0, sem1):
    cid = lax.axis_index("c")
    sid = lax.axis_index("s")
    wid = sid * NC + cid
    base = wid * BPW

    for c in range(NCH):
        pltpu.sync_copy(hi_hbm.at[pl.ds(base + c * CH, CH)], hi_v.at[c])
        pltpu.sync_copy(ri_hbm.at[pl.ds(base + c * CH, CH)], ri_v.at[c])
        pltpu.sync_copy(ti_hbm.at[pl.ds(base + c * CH, CH)], ti_v.at[c])

    bufs = [(hb0, rb0, tb0, sem0), (hb1, rb1, tb1, sem1)]

    def start(c):
        hb, rb, tb, sem = bufs[c % 2]
        return (
            pltpu.async_copy(ent_hbm.at[hi_v.at[c]], hb, sem),
            pltpu.async_copy(rel_hbm.at[ri_v.at[c]], rb, sem),
            pltpu.async_copy(ent_hbm.at[ti_v.at[c]], tb, sem),
        )

    pending = start(0)
    for c in range(NCH):
        for hdl in pending:
            hdl.wait()
        if c + 1 < NCH:
            pending = start(c + 1)
        hb, rb, tb, _ = bufs[c % 2]
        for g in range(CH // L):
            rows = lax.iota(jnp.int32, L) + (g * L)

            def dbody(d, acc, hb=hb, rb=rb, tb=tb, rows=rows):
                cols = jnp.full((L,), d, jnp.int32)
                h = plsc.load_gather(hb, [rows, cols])
                r = plsc.load_gather(rb, [rows, cols])
                t = plsc.load_gather(tb, [rows, cols])
                return acc + jnp.abs(h + r - t)

            acc = lax.fori_loop(0, DIM, dbody, jnp.zeros((L,), jnp.float32),
                                unroll=4)
            ob[pl.ds(c * CH + g * L, L)] = GAMMA - acc

    pltpu.sync_copy(ob, out_hbm.at[pl.ds(base, BPW)])


def kernel(sample, entity_embedding, relation_embedding):
    hi = sample[:, 0]
    ri = sample[:, 1]
    ti = sample[:, 2]
    score = _sc_score(hi, ri, ti, entity_embedding, relation_embedding)
    return score[:, None]


# speedup vs baseline: 2.9715x; 2.9715x over previous
import functools

import jax
import jax.numpy as jnp
from jax import lax
from jax.experimental import pallas as pl
from jax.experimental.pallas import tpu as pltpu
from jax.experimental.pallas import tpu_sc as plsc

B = 16384
DIM = 128
GAMMA = 12.0

NC = 2
NS = 16
L = 16
NW = NC * NS
BPW = B // NW
CH = 128
NCH = BPW // CH

_mesh = plsc.VectorSubcoreMesh(core_axis_name="c", subcore_axis_name="s")


@functools.partial(
    pl.kernel,
    mesh=_mesh,
    out_type=jax.ShapeDtypeStruct((B,), jnp.float32),
    scratch_types=[
        pltpu.VMEM((NCH, CH), jnp.int32),
        pltpu.VMEM((NCH, CH), jnp.int32),
        pltpu.VMEM((NCH, CH), jnp.int32),
        pltpu.VMEM((CH, DIM), jnp.float32),
        pltpu.VMEM((CH, DIM), jnp.float32),
        pltpu.VMEM((CH, DIM), jnp.float32),
        pltpu.VMEM((CH, DIM), jnp.float32),
        pltpu.VMEM((CH, DIM), jnp.float32),
        pltpu.VMEM((CH, DIM), jnp.float32),
        pltpu.VMEM((BPW,), jnp.float32),
        pltpu.SemaphoreType.DMA,
        pltpu.SemaphoreType.DMA,
    ],
    compiler_params=pltpu.CompilerParams(needs_layout_passes=False),
)
def _sc_score(hi_hbm, ri_hbm, ti_hbm, ent_hbm, rel_hbm, out_hbm,
              hi_v, ri_v, ti_v, hb0, hb1, rb0, rb1, tb0, tb1, ob,
              sem0, sem1):
    cid = lax.axis_index("c")
    sid = lax.axis_index("s")
    wid = sid * NC + cid
    base = wid * BPW

    for c in range(NCH):
        pltpu.sync_copy(hi_hbm.at[pl.ds(base + c * CH, CH)], hi_v.at[c])
        pltpu.sync_copy(ri_hbm.at[pl.ds(base + c * CH, CH)], ri_v.at[c])
        pltpu.sync_copy(ti_hbm.at[pl.ds(base + c * CH, CH)], ti_v.at[c])

    bufs = [(hb0, rb0, tb0, sem0), (hb1, rb1, tb1, sem1)]

    def start(c):
        hb, rb, tb, sem = bufs[c % 2]
        return (
            pltpu.async_copy(ent_hbm.at[hi_v.at[c]], hb, sem),
            pltpu.async_copy(rel_hbm.at[ri_v.at[c]], rb, sem),
            pltpu.async_copy(ent_hbm.at[ti_v.at[c]], tb, sem),
        )

    pending = start(0)
    for c in range(NCH):
        for hdl in pending:
            hdl.wait()
        if c + 1 < NCH:
            pending = start(c + 1)
        hb, rb, tb, _ = bufs[c % 2]
        for g in range(CH // L):
            lanes = lax.iota(jnp.int32, L)
            rows = lanes + (g * L)

            def dbody(d, acc, hb=hb, rb=rb, tb=tb, rows=rows, lanes=lanes):
                cols = (lanes + d) & (DIM - 1)
                h = plsc.load_gather(hb, [rows, cols])
                r = plsc.load_gather(rb, [rows, cols])
                t = plsc.load_gather(tb, [rows, cols])
                return acc + jnp.abs(h + r - t)

            acc = lax.fori_loop(0, DIM, dbody, jnp.zeros((L,), jnp.float32),
                                unroll=4)
            ob[pl.ds(c * CH + g * L, L)] = GAMMA - acc

    pltpu.sync_copy(ob, out_hbm.at[pl.ds(base, BPW)])


def kernel(sample, entity_embedding, relation_embedding):
    hi = sample[:, 0]
    ri = sample[:, 1]
    ti = sample[:, 2]
    score = _sc_score(hi, ri, ti, entity_embedding, relation_embedding)
    return score[:, None]


# speedup vs baseline: 3.2969x; 1.1095x over previous
import functools

import jax
import jax.numpy as jnp
from jax import lax
from jax.experimental import pallas as pl
from jax.experimental.pallas import tpu as pltpu
from jax.experimental.pallas import tpu_sc as plsc

B = 16384
DIM = 128
GAMMA = 12.0

NC = 2
NS = 16
L = 16
NW = NC * NS
BPW = B // NW
CH = 128
NCH = BPW // CH

_mesh = plsc.VectorSubcoreMesh(core_axis_name="c", subcore_axis_name="s")


@functools.partial(
    pl.kernel,
    mesh=_mesh,
    out_type=jax.ShapeDtypeStruct((B,), jnp.float32),
    scratch_types=[
        pltpu.VMEM((BPW,), jnp.int32),
        pltpu.VMEM((BPW,), jnp.int32),
        pltpu.VMEM((BPW,), jnp.int32),
        pltpu.VMEM((CH, DIM), jnp.float32),
        pltpu.VMEM((CH, DIM), jnp.float32),
        pltpu.VMEM((CH, DIM), jnp.float32),
        pltpu.VMEM((CH, DIM), jnp.float32),
        pltpu.VMEM((CH, DIM), jnp.float32),
        pltpu.VMEM((CH, DIM), jnp.float32),
        pltpu.VMEM((BPW,), jnp.float32),
        pltpu.SemaphoreType.DMA,
        pltpu.SemaphoreType.DMA,
    ],
    compiler_params=pltpu.CompilerParams(needs_layout_passes=False),
)
def _sc_score(hi_hbm, ri_hbm, ti_hbm, ent_hbm, rel_hbm, out_hbm,
              hi_v, ri_v, ti_v, hb0, hb1, rb0, rb1, tb0, tb1, ob,
              sem0, sem1):
    cid = lax.axis_index("c")
    sid = lax.axis_index("s")
    wid = sid * NC + cid
    base = wid * BPW

    for hdl in (
        pltpu.async_copy(hi_hbm.at[pl.ds(base, BPW)], hi_v, sem0),
        pltpu.async_copy(ri_hbm.at[pl.ds(base, BPW)], ri_v, sem0),
        pltpu.async_copy(ti_hbm.at[pl.ds(base, BPW)], ti_v, sem0),
    ):
        hdl.wait()

    bufs = [(hb0, rb0, tb0, sem0), (hb1, rb1, tb1, sem1)]

    def start(c):
        hb, rb, tb, sem = bufs[c % 2]
        sl = pl.ds(c * CH, CH)
        return (
            pltpu.async_copy(ent_hbm.at[hi_v.at[sl]], hb, sem),
            pltpu.async_copy(rel_hbm.at[ri_v.at[sl]], rb, sem),
            pltpu.async_copy(ent_hbm.at[ti_v.at[sl]], tb, sem),
        )

    UNR = 4
    pending = start(0)
    for c in range(NCH):
        for hdl in pending:
            hdl.wait()
        if c + 1 < NCH:
            pending = start(c + 1)
        hb, rb, tb, _ = bufs[c % 2]
        for g in range(CH // L):
            lanes = lax.iota(jnp.int32, L)
            rows = lanes + (g * L)

            def dbody(i, accs, hb=hb, rb=rb, tb=tb, rows=rows, lanes=lanes):
                out = []
                for k in range(UNR):
                    d = i * UNR + k
                    cols = (lanes + d) & (DIM - 1)
                    h = plsc.load_gather(hb, [rows, cols])
                    r = plsc.load_gather(rb, [rows, cols])
                    t = plsc.load_gather(tb, [rows, cols])
                    out.append(accs[k] + jnp.abs(h + r - t))
                return tuple(out)

            zero = jnp.zeros((L,), jnp.float32)
            accs = lax.fori_loop(0, DIM // UNR, dbody, (zero,) * UNR)
            acc = (accs[0] + accs[1]) + (accs[2] + accs[3])
            ob[pl.ds(c * CH + g * L, L)] = GAMMA - acc

    pltpu.sync_copy(ob, out_hbm.at[pl.ds(base, BPW)])


def kernel(sample, entity_embedding, relation_embedding):
    hi = sample[:, 0]
    ri = sample[:, 1]
    ti = sample[:, 2]
    score = _sc_score(hi, ri, ti, entity_embedding, relation_embedding)
    return score[:, None]
